# Initial kernel scaffold; baseline (speedup 1.0000x reference)
import jax
import jax.numpy as jnp
from jax.experimental import pallas as pl


def kernel(node_feature, edge_index, edge_feature, batch, params):
    raise NotImplementedError("write your pallas kernel here")


# speedup vs baseline: 2.3505x; 2.3505x over previous
import functools

import jax
import jax.numpy as jnp
from jax import lax
from jax.experimental import pallas as pl
from jax.experimental.pallas import tpu as pltpu
from jax.experimental.pallas import tpu_sc as plsc

N = 10000
E = 320000
D = 128
G = 16
HALF = 64
NUM_LAYERS = 2
EPS = 1e-07

NSUB = 16
EDGES_PER_TILE = E // NSUB
CHUNK = 80
STEPS = EDGES_PER_TILE // CHUNK
IBLK = 50
NBLK = STEPS // IBLK
ROW_STRIPE = 632
ROW_TAIL = N - (NSUB - 1) * ROW_STRIPE


def _per_tile_rows(s, f):
    @pl.when(s < NSUB - 1)
    def _():
        f(pl.ds(s * ROW_STRIPE, ROW_STRIPE))

    @pl.when(s == NSUB - 1)
    def _():
        f(pl.ds((NSUB - 1) * ROW_STRIPE, ROW_TAIL))


def _edge_body(src_hbm, dst_hbm, ea0_hbm, ea1_hbm, x0_hbm, x1_hbm, t_hbm,
               z_hbm,
               out0_hbm, out1_hbm,
               acc, srcv, dstv, xg, eav, comb, tv, sem):
    c = lax.axis_index("c")
    s = lax.axis_index("s")
    _per_tile_rows(s, lambda rows: pltpu.sync_copy(z_hbm.at[rows],
                                                   acc.at[rows]))
    pltpu.sync_copy(t_hbm, tv)
    plsc.subcore_barrier()
    tval = tv[...]
    ebase = s * EDGES_PER_TILE

    def run(x_hbm, ea_hbm):
        def blk(bk, carry0):
            isl = pl.ds(bk * IBLK, IBLK)
            pltpu.sync_copy(src_hbm.at[s].at[isl], srcv)
            pltpu.sync_copy(dst_hbm.at[s].at[isl], dstv)

            def step(j, carry):
                i = bk * IBLK + j
                pltpu.async_copy(x_hbm.at[srcv.at[j]], xg, sem).wait()
                pltpu.sync_copy(ea_hbm.at[pl.ds(ebase + i * CHUNK, CHUNK)],
                                eav)

                def row(r, carry2):
                    for g in range(HALF // 16):
                        sl = pl.ds(g * 16, 16)
                        a = xg[r, sl] + eav[r, sl]
                        m = jnp.maximum(a, 0.0) + EPS
                        e = jnp.exp(m * tval)
                        comb[r, sl] = e
                        comb[r, pl.ds(HALF + g * 16, 16)] = m * e
                    return carry2

                lax.fori_loop(0, CHUNK, row, 0)
                pltpu.sync_copy(comb, acc.at[dstv.at[j]], add=True)
                return carry

            lax.fori_loop(0, IBLK, step, 0)
            return carry0

        lax.fori_loop(0, NBLK, blk, 0)

    @pl.when(c == 0)
    def _():
        run(x0_hbm, ea0_hbm)

    @pl.when(c == 1)
    def _():
        run(x1_hbm, ea1_hbm)

    plsc.subcore_barrier()

    @pl.when(c == 0)
    def _():
        _per_tile_rows(s, lambda rows: pltpu.sync_copy(acc.at[rows],
                                                       out0_hbm.at[rows]))

    @pl.when(c == 1)
    def _():
        _per_tile_rows(s, lambda rows: pltpu.sync_copy(acc.at[rows],
                                                       out1_hbm.at[rows]))


_edge_call = pl.kernel(
    _edge_body,
    out_type=[jax.ShapeDtypeStruct((N, D), jnp.float32),
              jax.ShapeDtypeStruct((N, D), jnp.float32)],
    mesh=plsc.VectorSubcoreMesh(core_axis_name="c", subcore_axis_name="s"),
    scratch_types=[
        pltpu.VMEM_SHARED((N, D), jnp.float32),
        pltpu.VMEM((IBLK, CHUNK), jnp.int32),
        pltpu.VMEM((IBLK, CHUNK), jnp.int32),
        pltpu.VMEM((CHUNK, HALF), jnp.float32),
        pltpu.VMEM((CHUNK, HALF), jnp.float32),
        pltpu.VMEM((CHUNK, D), jnp.float32),
        pltpu.VMEM((16,), jnp.float32),
        pltpu.SemaphoreType.DMA,
    ],
    compiler_params=pltpu.CompilerParams(use_tc_tiling_on_sc=False),
)


def _node_body(x0_ref, x1_ref, a0_ref, a1_ref, w1_ref, b1_ref, g1_ref,
               be1_ref, w2_ref, b2_ref, sc_ref, y0_ref, y1_ref):
    x = jnp.concatenate([x0_ref[...], x1_ref[...]], axis=1)
    den = jnp.concatenate([a0_ref[:, :HALF], a1_ref[:, :HALF]], axis=1)
    num = jnp.concatenate([a0_ref[:, HALF:], a1_ref[:, HALF:]], axis=1)
    agg = num / (den + 1e-16)
    an = jnp.sqrt(jnp.sum(agg * agg, axis=1, keepdims=True))
    msg_n = agg / jnp.maximum(an, 1e-12)
    xn = jnp.sqrt(jnp.sum(x * x, axis=1, keepdims=True))
    h = x + sc_ref[0, 0] * msg_n * xn
    h2 = jnp.dot(h, w1_ref[...], preferred_element_type=jnp.float32)
    h2 = h2 + b1_ref[...]
    mu = jnp.mean(h2, axis=1, keepdims=True)
    var = jnp.mean((h2 - mu) ** 2, axis=1, keepdims=True)
    h2 = (h2 - mu) * lax.rsqrt(var + 1e-5) * g1_ref[...] + be1_ref[...]
    h2 = jnp.maximum(h2, 0.0)
    y = jnp.dot(h2, w2_ref[...], preferred_element_type=jnp.float32)
    y = y + b2_ref[...]
    y0_ref[...] = y[:, :HALF]
    y1_ref[...] = y[:, HALF:]


def _node_call(x0, x1, acc0, acc1, p):
    B = 2000
    return pl.pallas_call(
        _node_body,
        grid=(N // B,),
        in_specs=[
            pl.BlockSpec((B, HALF), lambda i: (i, 0)),
            pl.BlockSpec((B, HALF), lambda i: (i, 0)),
            pl.BlockSpec((B, D), lambda i: (i, 0)),
            pl.BlockSpec((B, D), lambda i: (i, 0)),
            pl.BlockSpec((D, 2 * D), lambda i: (0, 0)),
            pl.BlockSpec((1, 2 * D), lambda i: (0, 0)),
            pl.BlockSpec((1, 2 * D), lambda i: (0, 0)),
            pl.BlockSpec((1, 2 * D), lambda i: (0, 0)),
            pl.BlockSpec((2 * D, D), lambda i: (0, 0)),
            pl.BlockSpec((1, D), lambda i: (0, 0)),
            pl.BlockSpec(memory_space=pltpu.SMEM),
        ],
        out_specs=[pl.BlockSpec((B, HALF), lambda i: (i, 0)),
                   pl.BlockSpec((B, HALF), lambda i: (i, 0))],
        out_shape=[jax.ShapeDtypeStruct((N, HALF), jnp.float32),
                   jax.ShapeDtypeStruct((N, HALF), jnp.float32)],
    )(x0, x1, acc0, acc1, p['w1'], p['b1'].reshape(1, -1),
      p['g1'].reshape(1, -1), p['be1'].reshape(1, -1), p['w2'],
      p['b2'].reshape(1, -1), p['scale'].reshape(1, 1))


def _final_body(y0_ref, y1_ref, bb_ref, n1g_ref, n1b_ref, lw_ref, lb_ref,
                n2g_ref, n2b_ref, local_ref, gl_ref, gsum_ref, cnt_ref):
    i = pl.program_id(0)
    xf = jnp.concatenate([y0_ref[...], y1_ref[...]], axis=1)
    lm = jnp.mean(xf, axis=1, keepdims=True)
    lv = jnp.mean((xf - lm) ** 2, axis=1, keepdims=True)
    local = (xf - lm) * lax.rsqrt(lv + 1e-5) * n1g_ref[...] + n1b_ref[...]
    local_ref[...] = local
    oh = (bb_ref[...] == lax.broadcasted_iota(jnp.int32, (1, G), 1))
    oh = oh.astype(jnp.float32)
    ps = lax.dot_general(oh, local, (((0,), (0,)), ((), ())),
                         preferred_element_type=jnp.float32)
    pc = lax.dot_general(oh, jnp.ones_like(local), (((0,), (0,)), ((), ())),
                         preferred_element_type=jnp.float32)

    @pl.when(i == 0)
    def _():
        gsum_ref[...] = ps
        cnt_ref[...] = pc

    @pl.when(i > 0)
    def _():
        gsum_ref[...] += ps
        cnt_ref[...] += pc

    @pl.when(i == pl.num_programs(0) - 1)
    def _():
        gmean = gsum_ref[...] / jnp.maximum(cnt_ref[...], 1.0)
        gl = jnp.dot(gmean, lw_ref[...], preferred_element_type=jnp.float32)
        gl = gl + lb_ref[...]
        gm = jnp.mean(gl, axis=1, keepdims=True)
        gv = jnp.mean((gl - gm) ** 2, axis=1, keepdims=True)
        gl_ref[...] = (gl - gm) * lax.rsqrt(gv + 1e-5) * n2g_ref[...] \
            + n2b_ref[...]


def _final_call(y0, y1, batch2, params):
    B = 2000
    return pl.pallas_call(
        _final_body,
        grid=(N // B,),
        in_specs=[
            pl.BlockSpec((B, HALF), lambda i: (i, 0)),
            pl.BlockSpec((B, HALF), lambda i: (i, 0)),
            pl.BlockSpec((B, 1), lambda i: (i, 0)),
            pl.BlockSpec((1, D), lambda i: (0, 0)),
            pl.BlockSpec((1, D), lambda i: (0, 0)),
            pl.BlockSpec((D, D), lambda i: (0, 0)),
            pl.BlockSpec((1, D), lambda i: (0, 0)),
            pl.BlockSpec((1, D), lambda i: (0, 0)),
            pl.BlockSpec((1, D), lambda i: (0, 0)),
        ],
        out_specs=[pl.BlockSpec((B, D), lambda i: (i, 0)),
                   pl.BlockSpec((G, D), lambda i: (0, 0))],
        out_shape=[jax.ShapeDtypeStruct((N, D), jnp.float32),
                   jax.ShapeDtypeStruct((G, D), jnp.float32)],
        scratch_shapes=[pltpu.VMEM((G, D), jnp.float32),
                        pltpu.VMEM((G, D), jnp.float32)],
    )(y0, y1, batch2, params['n1_g'].reshape(1, -1),
      params['n1_b'].reshape(1, -1), params['lin_w'],
      params['lin_b'].reshape(1, -1), params['n2_g'].reshape(1, -1),
      params['n2_b'].reshape(1, -1))


def kernel(node_feature, edge_index, edge_feature, batch, params):
    ei3 = edge_index.astype(jnp.int32).reshape(2, NSUB, STEPS, CHUNK)
    src2, dst2 = ei3[0], ei3[1]
    x0 = node_feature[:, :HALF]
    x1 = node_feature[:, HALF:]
    ea0 = edge_feature[:, :HALF]
    ea1 = edge_feature[:, HALF:]
    zeros_nd = jnp.zeros((N, D), jnp.float32)
    batch2 = batch.astype(jnp.int32).reshape(N, 1)
    for l in range(NUM_LAYERS):
        p = params['layers'][l]
        t16 = jnp.full((16,), p['t'], jnp.float32)
        acc0, acc1 = _edge_call(src2, dst2, ea0, ea1, x0, x1, t16,
                                zeros_nd)
        x0, x1 = _node_call(x0, x1, acc0, acc1, p)
    local, gl = _final_call(x0, x1, batch2, params)
    return (local, gl)


# speedup vs baseline: 3.0764x; 1.3088x over previous
import functools

import jax
import jax.numpy as jnp
from jax import lax
from jax.experimental import pallas as pl
from jax.experimental.pallas import tpu as pltpu
from jax.experimental.pallas import tpu_sc as plsc

N = 10000
E = 320000
D = 128
G = 16
HALF = 64
NUM_LAYERS = 2
EPS = 1e-07

NSUB = 16
EDGES_PER_TILE = E // NSUB
CHUNK = 80
STEPS = EDGES_PER_TILE // CHUNK
IBLK = 50
NBLK = STEPS // IBLK
ROW_STRIPE = 632
ROW_TAIL = N - (NSUB - 1) * ROW_STRIPE


def _per_tile_rows(s, f):
    @pl.when(s < NSUB - 1)
    def _():
        f(pl.ds(s * ROW_STRIPE, ROW_STRIPE))

    @pl.when(s == NSUB - 1)
    def _():
        f(pl.ds((NSUB - 1) * ROW_STRIPE, ROW_TAIL))


def _edge_body(src_hbm, dst_hbm, ea0_hbm, ea1_hbm, x0_hbm, x1_hbm, t_hbm,
               z_hbm,
               out0_hbm, out1_hbm,
               acc, srcv, dstv, xg0, xg1, eav0, eav1, comb0, comb1, tv,
               gsem0, gsem1, esem0, esem1, ssem0, ssem1):
    c = lax.axis_index("c")
    s = lax.axis_index("s")
    _per_tile_rows(s, lambda rows: pltpu.sync_copy(z_hbm.at[rows],
                                                   acc.at[rows]))
    pltpu.sync_copy(t_hbm, tv)
    plsc.subcore_barrier()
    tval = tv[...]
    ebase = s * EDGES_PER_TILE

    def run(x_hbm, ea_hbm):
        xgs = (xg0, xg1)
        eavs = (eav0, eav1)
        combs = (comb0, comb1)
        gsems = (gsem0, gsem1)
        esems = (esem0, esem1)
        ssems = (ssem0, ssem1)

        def issue(bk, j, b):
            pltpu.async_copy(x_hbm.at[srcv.at[j]], xgs[b], gsems[b])
            off = ebase + (bk * IBLK + j) * CHUNK
            pltpu.async_copy(ea_hbm.at[pl.ds(off, CHUNK)], eavs[b], esems[b])

        def wait_in(b):
            pltpu.make_async_copy(x_hbm.at[srcv.at[0]], xgs[b],
                                  gsems[b]).wait()
            pltpu.make_async_copy(ea_hbm.at[pl.ds(ebase, CHUNK)], eavs[b],
                                  esems[b]).wait()

        def wait_sc(b):
            pltpu.make_async_copy(combs[b], acc.at[dstv.at[0]],
                                  ssems[b]).wait()

        def compute(b):
            def row(r, carry2):
                for g in range(HALF // 16):
                    sl = pl.ds(g * 16, 16)
                    a = xgs[b][r, sl] + eavs[b][r, sl]
                    m = jnp.maximum(a, 0.0) + EPS
                    e = jnp.exp(m * tval)
                    combs[b][r, sl] = e
                    combs[b][r, pl.ds(HALF + g * 16, 16)] = m * e
                return carry2

            lax.fori_loop(0, CHUNK, row, 0)

        def half_step(bk, q, j, b):
            wait_in(b)
            @pl.when(q > 0)
            def _():
                wait_sc(b)

            compute(b)
            pltpu.async_copy(combs[b], acc.at[dstv.at[j]], ssems[b],
                             add=True)

        Q = IBLK // 2

        def blk(bk, carry0):
            isl = pl.ds(bk * IBLK, IBLK)
            pltpu.sync_copy(src_hbm.at[s].at[isl], srcv)
            pltpu.sync_copy(dst_hbm.at[s].at[isl], dstv)
            issue(bk, 0, 0)

            def pair(q, carry):
                issue(bk, 2 * q + 1, 1)
                half_step(bk, q, 2 * q, 0)

                @pl.when(q < Q - 1)
                def _():
                    issue(bk, 2 * q + 2, 0)

                half_step(bk, q, 2 * q + 1, 1)
                return carry

            lax.fori_loop(0, Q, pair, 0)
            wait_sc(0)
            wait_sc(1)
            return carry0

        lax.fori_loop(0, NBLK, blk, 0)

    @pl.when(c == 0)
    def _():
        run(x0_hbm, ea0_hbm)

    @pl.when(c == 1)
    def _():
        run(x1_hbm, ea1_hbm)

    plsc.subcore_barrier()

    @pl.when(c == 0)
    def _():
        _per_tile_rows(s, lambda rows: pltpu.sync_copy(acc.at[rows],
                                                       out0_hbm.at[rows]))

    @pl.when(c == 1)
    def _():
        _per_tile_rows(s, lambda rows: pltpu.sync_copy(acc.at[rows],
                                                       out1_hbm.at[rows]))


_edge_call = pl.kernel(
    _edge_body,
    out_type=[jax.ShapeDtypeStruct((N, D), jnp.float32),
              jax.ShapeDtypeStruct((N, D), jnp.float32)],
    mesh=plsc.VectorSubcoreMesh(core_axis_name="c", subcore_axis_name="s"),
    scratch_types=[
        pltpu.VMEM_SHARED((N, D), jnp.float32),
        pltpu.VMEM((IBLK, CHUNK), jnp.int32),
        pltpu.VMEM((IBLK, CHUNK), jnp.int32),
        pltpu.VMEM((CHUNK, HALF), jnp.float32),
        pltpu.VMEM((CHUNK, HALF), jnp.float32),
        pltpu.VMEM((CHUNK, HALF), jnp.float32),
        pltpu.VMEM((CHUNK, HALF), jnp.float32),
        pltpu.VMEM((CHUNK, D), jnp.float32),
        pltpu.VMEM((CHUNK, D), jnp.float32),
        pltpu.VMEM((16,), jnp.float32),
        pltpu.SemaphoreType.DMA,
        pltpu.SemaphoreType.DMA,
        pltpu.SemaphoreType.DMA,
        pltpu.SemaphoreType.DMA,
        pltpu.SemaphoreType.DMA,
        pltpu.SemaphoreType.DMA,
    ],
    compiler_params=pltpu.CompilerParams(use_tc_tiling_on_sc=False),
)


def _node_body(x0_ref, x1_ref, a0_ref, a1_ref, w1_ref, b1_ref, g1_ref,
               be1_ref, w2_ref, b2_ref, sc_ref, y0_ref, y1_ref):
    x = jnp.concatenate([x0_ref[...], x1_ref[...]], axis=1)
    den = jnp.concatenate([a0_ref[:, :HALF], a1_ref[:, :HALF]], axis=1)
    num = jnp.concatenate([a0_ref[:, HALF:], a1_ref[:, HALF:]], axis=1)
    agg = num / (den + 1e-16)
    an = jnp.sqrt(jnp.sum(agg * agg, axis=1, keepdims=True))
    msg_n = agg / jnp.maximum(an, 1e-12)
    xn = jnp.sqrt(jnp.sum(x * x, axis=1, keepdims=True))
    h = x + sc_ref[0, 0] * msg_n * xn
    h2 = jnp.dot(h, w1_ref[...], preferred_element_type=jnp.float32)
    h2 = h2 + b1_ref[...]
    mu = jnp.mean(h2, axis=1, keepdims=True)
    var = jnp.mean((h2 - mu) ** 2, axis=1, keepdims=True)
    h2 = (h2 - mu) * lax.rsqrt(var + 1e-5) * g1_ref[...] + be1_ref[...]
    h2 = jnp.maximum(h2, 0.0)
    y = jnp.dot(h2, w2_ref[...], preferred_element_type=jnp.float32)
    y = y + b2_ref[...]
    y0_ref[...] = y[:, :HALF]
    y1_ref[...] = y[:, HALF:]


def _node_call(x0, x1, acc0, acc1, p):
    B = 2000
    return pl.pallas_call(
        _node_body,
        grid=(N // B,),
        in_specs=[
            pl.BlockSpec((B, HALF), lambda i: (i, 0)),
            pl.BlockSpec((B, HALF), lambda i: (i, 0)),
            pl.BlockSpec((B, D), lambda i: (i, 0)),
            pl.BlockSpec((B, D), lambda i: (i, 0)),
            pl.BlockSpec((D, 2 * D), lambda i: (0, 0)),
            pl.BlockSpec((1, 2 * D), lambda i: (0, 0)),
            pl.BlockSpec((1, 2 * D), lambda i: (0, 0)),
            pl.BlockSpec((1, 2 * D), lambda i: (0, 0)),
            pl.BlockSpec((2 * D, D), lambda i: (0, 0)),
            pl.BlockSpec((1, D), lambda i: (0, 0)),
            pl.BlockSpec(memory_space=pltpu.SMEM),
        ],
        out_specs=[pl.BlockSpec((B, HALF), lambda i: (i, 0)),
                   pl.BlockSpec((B, HALF), lambda i: (i, 0))],
        out_shape=[jax.ShapeDtypeStruct((N, HALF), jnp.float32),
                   jax.ShapeDtypeStruct((N, HALF), jnp.float32)],
    )(x0, x1, acc0, acc1, p['w1'], p['b1'].reshape(1, -1),
      p['g1'].reshape(1, -1), p['be1'].reshape(1, -1), p['w2'],
      p['b2'].reshape(1, -1), p['scale'].reshape(1, 1))


def _final_body(y0_ref, y1_ref, bb_ref, n1g_ref, n1b_ref, lw_ref, lb_ref,
                n2g_ref, n2b_ref, local_ref, gl_ref, gsum_ref, cnt_ref):
    i = pl.program_id(0)
    xf = jnp.concatenate([y0_ref[...], y1_ref[...]], axis=1)
    lm = jnp.mean(xf, axis=1, keepdims=True)
    lv = jnp.mean((xf - lm) ** 2, axis=1, keepdims=True)
    local = (xf - lm) * lax.rsqrt(lv + 1e-5) * n1g_ref[...] + n1b_ref[...]
    local_ref[...] = local
    oh = (bb_ref[...] == lax.broadcasted_iota(jnp.int32, (1, G), 1))
    oh = oh.astype(jnp.float32)
    ps = lax.dot_general(oh, local, (((0,), (0,)), ((), ())),
                         preferred_element_type=jnp.float32)
    pc = lax.dot_general(oh, jnp.ones_like(local), (((0,), (0,)), ((), ())),
                         preferred_element_type=jnp.float32)

    @pl.when(i == 0)
    def _():
        gsum_ref[...] = ps
        cnt_ref[...] = pc

    @pl.when(i > 0)
    def _():
        gsum_ref[...] += ps
        cnt_ref[...] += pc

    @pl.when(i == pl.num_programs(0) - 1)
    def _():
        gmean = gsum_ref[...] / jnp.maximum(cnt_ref[...], 1.0)
        gl = jnp.dot(gmean, lw_ref[...], preferred_element_type=jnp.float32)
        gl = gl + lb_ref[...]
        gm = jnp.mean(gl, axis=1, keepdims=True)
        gv = jnp.mean((gl - gm) ** 2, axis=1, keepdims=True)
        gl_ref[...] = (gl - gm) * lax.rsqrt(gv + 1e-5) * n2g_ref[...] \
            + n2b_ref[...]


def _final_call(y0, y1, batch2, params):
    B = 2000
    return pl.pallas_call(
        _final_body,
        grid=(N // B,),
        in_specs=[
            pl.BlockSpec((B, HALF), lambda i: (i, 0)),
            pl.BlockSpec((B, HALF), lambda i: (i, 0)),
            pl.BlockSpec((B, 1), lambda i: (i, 0)),
            pl.BlockSpec((1, D), lambda i: (0, 0)),
            pl.BlockSpec((1, D), lambda i: (0, 0)),
            pl.BlockSpec((D, D), lambda i: (0, 0)),
            pl.BlockSpec((1, D), lambda i: (0, 0)),
            pl.BlockSpec((1, D), lambda i: (0, 0)),
            pl.BlockSpec((1, D), lambda i: (0, 0)),
        ],
        out_specs=[pl.BlockSpec((B, D), lambda i: (i, 0)),
                   pl.BlockSpec((G, D), lambda i: (0, 0))],
        out_shape=[jax.ShapeDtypeStruct((N, D), jnp.float32),
                   jax.ShapeDtypeStruct((G, D), jnp.float32)],
        scratch_shapes=[pltpu.VMEM((G, D), jnp.float32),
                        pltpu.VMEM((G, D), jnp.float32)],
    )(y0, y1, batch2, params['n1_g'].reshape(1, -1),
      params['n1_b'].reshape(1, -1), params['lin_w'],
      params['lin_b'].reshape(1, -1), params['n2_g'].reshape(1, -1),
      params['n2_b'].reshape(1, -1))


def kernel(node_feature, edge_index, edge_feature, batch, params):
    ei3 = edge_index.astype(jnp.int32).reshape(2, NSUB, STEPS, CHUNK)
    src2, dst2 = ei3[0], ei3[1]
    x0 = node_feature[:, :HALF]
    x1 = node_feature[:, HALF:]
    ea0 = edge_feature[:, :HALF]
    ea1 = edge_feature[:, HALF:]
    zeros_nd = jnp.zeros((N, D), jnp.float32)
    batch2 = batch.astype(jnp.int32).reshape(N, 1)
    for l in range(NUM_LAYERS):
        p = params['layers'][l]
        t16 = jnp.full((16,), p['t'], jnp.float32)
        acc0, acc1 = _edge_call(src2, dst2, ea0, ea1, x0, x1, t16,
                                zeros_nd)
        x0, x1 = _node_call(x0, x1, acc0, acc1, p)
    local, gl = _final_call(x0, x1, batch2, params)
    return (local, gl)


# speedup vs baseline: 9.0369x; 2.9375x over previous
import functools

import jax
import jax.numpy as jnp
from jax import lax
from jax.experimental import pallas as pl
from jax.experimental.pallas import tpu as pltpu
from jax.experimental.pallas import tpu_sc as plsc

N = 10000
E = 320000
D = 128
G = 16
HALF = 64
NUM_LAYERS = 2
EPS = 1e-07

NSUB = 16
EDGES_PER_TILE = E // NSUB
CHUNK = 80
STEPS = EDGES_PER_TILE // CHUNK
IBLK = 50
NBLK = STEPS // IBLK
ROW_STRIPE = 632
ROW_TAIL = N - (NSUB - 1) * ROW_STRIPE


def _per_tile_rows(s, f):
    @pl.when(s < NSUB - 1)
    def _():
        f(pl.ds(s * ROW_STRIPE, ROW_STRIPE))

    @pl.when(s == NSUB - 1)
    def _():
        f(pl.ds((NSUB - 1) * ROW_STRIPE, ROW_TAIL))


def _edge_body(src_hbm, dst_hbm, ea0_hbm, ea1_hbm, x0_hbm, x1_hbm, t_hbm,
               z_hbm,
               out0_hbm, out1_hbm,
               acc, srcv, dstv, xg0, xg1, eav0, eav1, comb0, comb1, tv,
               gsem0, gsem1, esem0, esem1, ssem0, ssem1):
    c = lax.axis_index("c")
    s = lax.axis_index("s")
    _per_tile_rows(s, lambda rows: pltpu.sync_copy(z_hbm.at[rows],
                                                   acc.at[rows]))
    pltpu.sync_copy(t_hbm, tv)
    plsc.subcore_barrier()
    tval = tv[...]
    ebase = s * EDGES_PER_TILE

    def run(x_hbm, ea_hbm):
        xgs = (xg0, xg1)
        eavs = (eav0, eav1)
        combs = (comb0, comb1)
        gsems = (gsem0, gsem1)
        esems = (esem0, esem1)
        ssems = (ssem0, ssem1)

        def issue(bk, j, b):
            pltpu.async_copy(x_hbm.at[srcv.at[j]], xgs[b], gsems[b])
            off = ebase + (bk * IBLK + j) * CHUNK
            pltpu.async_copy(ea_hbm.at[pl.ds(off, CHUNK)], eavs[b], esems[b])

        def wait_in(b):
            pltpu.make_async_copy(x_hbm.at[srcv.at[0]], xgs[b],
                                  gsems[b]).wait()
            pltpu.make_async_copy(ea_hbm.at[pl.ds(ebase, CHUNK)], eavs[b],
                                  esems[b]).wait()

        def wait_sc(b):
            pltpu.make_async_copy(combs[b], acc.at[dstv.at[0]],
                                  ssems[b]).wait()

        def compute(b):
            @plsc.parallel_loop(0, CHUNK, step=1, unroll=8)
            def row(r):
                for g in range(HALF // 16):
                    sl = pl.ds(g * 16, 16)
                    a = xgs[b][r, sl] + eavs[b][r, sl]
                    m = jnp.maximum(a, 0.0) + EPS
                    e = jnp.exp(m * tval)
                    combs[b][r, sl] = e
                    combs[b][r, pl.ds(HALF + g * 16, 16)] = m * e

        def half_step(bk, q, j, b):
            wait_in(b)
            @pl.when(q > 0)
            def _():
                wait_sc(b)

            compute(b)
            pltpu.async_copy(combs[b], acc.at[dstv.at[j]], ssems[b],
                             add=True)

        Q = IBLK // 2

        def blk(bk, carry0):
            isl = pl.ds(bk * IBLK, IBLK)
            pltpu.sync_copy(src_hbm.at[s].at[isl], srcv)
            pltpu.sync_copy(dst_hbm.at[s].at[isl], dstv)
            issue(bk, 0, 0)

            def pair(q, carry):
                issue(bk, 2 * q + 1, 1)
                half_step(bk, q, 2 * q, 0)

                @pl.when(q < Q - 1)
                def _():
                    issue(bk, 2 * q + 2, 0)

                half_step(bk, q, 2 * q + 1, 1)
                return carry

            lax.fori_loop(0, Q, pair, 0)
            wait_sc(0)
            wait_sc(1)
            return carry0

        lax.fori_loop(0, NBLK, blk, 0)

    @pl.when(c == 0)
    def _():
        run(x0_hbm, ea0_hbm)

    @pl.when(c == 1)
    def _():
        run(x1_hbm, ea1_hbm)

    plsc.subcore_barrier()

    @pl.when(c == 0)
    def _():
        _per_tile_rows(s, lambda rows: pltpu.sync_copy(acc.at[rows],
                                                       out0_hbm.at[rows]))

    @pl.when(c == 1)
    def _():
        _per_tile_rows(s, lambda rows: pltpu.sync_copy(acc.at[rows],
                                                       out1_hbm.at[rows]))


_edge_call = pl.kernel(
    _edge_body,
    out_type=[jax.ShapeDtypeStruct((N, D), jnp.float32),
              jax.ShapeDtypeStruct((N, D), jnp.float32)],
    mesh=plsc.VectorSubcoreMesh(core_axis_name="c", subcore_axis_name="s"),
    scratch_types=[
        pltpu.VMEM_SHARED((N, D), jnp.float32),
        pltpu.VMEM((IBLK, CHUNK), jnp.int32),
        pltpu.VMEM((IBLK, CHUNK), jnp.int32),
        pltpu.VMEM((CHUNK, HALF), jnp.float32),
        pltpu.VMEM((CHUNK, HALF), jnp.float32),
        pltpu.VMEM((CHUNK, HALF), jnp.float32),
        pltpu.VMEM((CHUNK, HALF), jnp.float32),
        pltpu.VMEM((CHUNK, D), jnp.float32),
        pltpu.VMEM((CHUNK, D), jnp.float32),
        pltpu.VMEM((16,), jnp.float32),
        pltpu.SemaphoreType.DMA,
        pltpu.SemaphoreType.DMA,
        pltpu.SemaphoreType.DMA,
        pltpu.SemaphoreType.DMA,
        pltpu.SemaphoreType.DMA,
        pltpu.SemaphoreType.DMA,
    ],
    compiler_params=pltpu.CompilerParams(use_tc_tiling_on_sc=False),
)


def _node_body(x0_ref, x1_ref, a0_ref, a1_ref, w1_ref, b1_ref, g1_ref,
               be1_ref, w2_ref, b2_ref, sc_ref, y0_ref, y1_ref):
    x = jnp.concatenate([x0_ref[...], x1_ref[...]], axis=1)
    den = jnp.concatenate([a0_ref[:, :HALF], a1_ref[:, :HALF]], axis=1)
    num = jnp.concatenate([a0_ref[:, HALF:], a1_ref[:, HALF:]], axis=1)
    agg = num / (den + 1e-16)
    an = jnp.sqrt(jnp.sum(agg * agg, axis=1, keepdims=True))
    msg_n = agg / jnp.maximum(an, 1e-12)
    xn = jnp.sqrt(jnp.sum(x * x, axis=1, keepdims=True))
    h = x + sc_ref[0, 0] * msg_n * xn
    h2 = jnp.dot(h, w1_ref[...], preferred_element_type=jnp.float32)
    h2 = h2 + b1_ref[...]
    mu = jnp.mean(h2, axis=1, keepdims=True)
    var = jnp.mean((h2 - mu) ** 2, axis=1, keepdims=True)
    h2 = (h2 - mu) * lax.rsqrt(var + 1e-5) * g1_ref[...] + be1_ref[...]
    h2 = jnp.maximum(h2, 0.0)
    y = jnp.dot(h2, w2_ref[...], preferred_element_type=jnp.float32)
    y = y + b2_ref[...]
    y0_ref[...] = y[:, :HALF]
    y1_ref[...] = y[:, HALF:]


def _node_call(x0, x1, acc0, acc1, p):
    B = 2000
    return pl.pallas_call(
        _node_body,
        grid=(N // B,),
        in_specs=[
            pl.BlockSpec((B, HALF), lambda i: (i, 0)),
            pl.BlockSpec((B, HALF), lambda i: (i, 0)),
            pl.BlockSpec((B, D), lambda i: (i, 0)),
            pl.BlockSpec((B, D), lambda i: (i, 0)),
            pl.BlockSpec((D, 2 * D), lambda i: (0, 0)),
            pl.BlockSpec((1, 2 * D), lambda i: (0, 0)),
            pl.BlockSpec((1, 2 * D), lambda i: (0, 0)),
            pl.BlockSpec((1, 2 * D), lambda i: (0, 0)),
            pl.BlockSpec((2 * D, D), lambda i: (0, 0)),
            pl.BlockSpec((1, D), lambda i: (0, 0)),
            pl.BlockSpec(memory_space=pltpu.SMEM),
        ],
        out_specs=[pl.BlockSpec((B, HALF), lambda i: (i, 0)),
                   pl.BlockSpec((B, HALF), lambda i: (i, 0))],
        out_shape=[jax.ShapeDtypeStruct((N, HALF), jnp.float32),
                   jax.ShapeDtypeStruct((N, HALF), jnp.float32)],
    )(x0, x1, acc0, acc1, p['w1'], p['b1'].reshape(1, -1),
      p['g1'].reshape(1, -1), p['be1'].reshape(1, -1), p['w2'],
      p['b2'].reshape(1, -1), p['scale'].reshape(1, 1))


def _final_body(y0_ref, y1_ref, bb_ref, n1g_ref, n1b_ref, lw_ref, lb_ref,
                n2g_ref, n2b_ref, local_ref, gl_ref, gsum_ref, cnt_ref):
    i = pl.program_id(0)
    xf = jnp.concatenate([y0_ref[...], y1_ref[...]], axis=1)
    lm = jnp.mean(xf, axis=1, keepdims=True)
    lv = jnp.mean((xf - lm) ** 2, axis=1, keepdims=True)
    local = (xf - lm) * lax.rsqrt(lv + 1e-5) * n1g_ref[...] + n1b_ref[...]
    local_ref[...] = local
    oh = (bb_ref[...] == lax.broadcasted_iota(jnp.int32, (1, G), 1))
    oh = oh.astype(jnp.float32)
    ps = lax.dot_general(oh, local, (((0,), (0,)), ((), ())),
                         preferred_element_type=jnp.float32)
    pc = lax.dot_general(oh, jnp.ones_like(local), (((0,), (0,)), ((), ())),
                         preferred_element_type=jnp.float32)

    @pl.when(i == 0)
    def _():
        gsum_ref[...] = ps
        cnt_ref[...] = pc

    @pl.when(i > 0)
    def _():
        gsum_ref[...] += ps
        cnt_ref[...] += pc

    @pl.when(i == pl.num_programs(0) - 1)
    def _():
        gmean = gsum_ref[...] / jnp.maximum(cnt_ref[...], 1.0)
        gl = jnp.dot(gmean, lw_ref[...], preferred_element_type=jnp.float32)
        gl = gl + lb_ref[...]
        gm = jnp.mean(gl, axis=1, keepdims=True)
        gv = jnp.mean((gl - gm) ** 2, axis=1, keepdims=True)
        gl_ref[...] = (gl - gm) * lax.rsqrt(gv + 1e-5) * n2g_ref[...] \
            + n2b_ref[...]


def _final_call(y0, y1, batch2, params):
    B = 2000
    return pl.pallas_call(
        _final_body,
        grid=(N // B,),
        in_specs=[
            pl.BlockSpec((B, HALF), lambda i: (i, 0)),
            pl.BlockSpec((B, HALF), lambda i: (i, 0)),
            pl.BlockSpec((B, 1), lambda i: (i, 0)),
            pl.BlockSpec((1, D), lambda i: (0, 0)),
            pl.BlockSpec((1, D), lambda i: (0, 0)),
            pl.BlockSpec((D, D), lambda i: (0, 0)),
            pl.BlockSpec((1, D), lambda i: (0, 0)),
            pl.BlockSpec((1, D), lambda i: (0, 0)),
            pl.BlockSpec((1, D), lambda i: (0, 0)),
        ],
        out_specs=[pl.BlockSpec((B, D), lambda i: (i, 0)),
                   pl.BlockSpec((G, D), lambda i: (0, 0))],
        out_shape=[jax.ShapeDtypeStruct((N, D), jnp.float32),
                   jax.ShapeDtypeStruct((G, D), jnp.float32)],
        scratch_shapes=[pltpu.VMEM((G, D), jnp.float32),
                        pltpu.VMEM((G, D), jnp.float32)],
    )(y0, y1, batch2, params['n1_g'].reshape(1, -1),
      params['n1_b'].reshape(1, -1), params['lin_w'],
      params['lin_b'].reshape(1, -1), params['n2_g'].reshape(1, -1),
      params['n2_b'].reshape(1, -1))


def kernel(node_feature, edge_index, edge_feature, batch, params):
    ei3 = edge_index.astype(jnp.int32).reshape(2, NSUB, STEPS, CHUNK)
    src2, dst2 = ei3[0], ei3[1]
    x0 = node_feature[:, :HALF]
    x1 = node_feature[:, HALF:]
    ea0 = edge_feature[:, :HALF]
    ea1 = edge_feature[:, HALF:]
    zeros_nd = jnp.zeros((N, D), jnp.float32)
    batch2 = batch.astype(jnp.int32).reshape(N, 1)
    for l in range(NUM_LAYERS):
        p = params['layers'][l]
        t16 = jnp.full((16,), p['t'], jnp.float32)
        acc0, acc1 = _edge_call(src2, dst2, ea0, ea1, x0, x1, t16,
                                zeros_nd)
        x0, x1 = _node_call(x0, x1, acc0, acc1, p)
    local, gl = _final_call(x0, x1, batch2, params)
    return (local, gl)
